# Initial kernel scaffold; baseline (speedup 1.0000x reference)
import functools

import jax
import jax.numpy as jnp
from jax import lax
from jax.experimental import pallas as pl
from jax.experimental.pallas import tpu as pltpu
from jax.experimental.pallas import tpu_sc as plsc

N = 10000
K = 32
D = 128
O = 128
C = 128

NW = 32
NPW = 320
NPAD = NW * NPW
CN = 4
CHUNK = CN * K
NCH = NPW // CN
NV = O // 16

BN = 256


def _film_body(cond_ref, vt_ref, g_ref, b2_ref, x_ref, wt_ref, bf_ref,
               o_ref, gb_ref):
  @pl.when(pl.program_id(0) == 0)
  def _():
    vt = vt_ref[...]
    ssq = jnp.sum(vt * vt, axis=0, keepdims=True)
    scale = g_ref[...] * lax.rsqrt(ssq)
    gb_ref[...] = (
        jnp.dot(cond_ref[...], vt, preferred_element_type=jnp.float32,
                precision=lax.Precision.HIGHEST) * scale + b2_ref[...])

  h = jnp.dot(x_ref[...], wt_ref[...], preferred_element_type=jnp.float32,
              precision=lax.Precision.HIGHEST) + bf_ref[...]
  mu = jnp.mean(h, axis=1, keepdims=True)
  hc = h - mu
  var = jnp.mean(hc * hc, axis=1, keepdims=True)
  hn = hc * lax.rsqrt(var + 1e-5)
  gamma = gb_ref[:, :O] + 1.0
  beta = gb_ref[:, O:]
  o_ref[...] = jnp.maximum(hn * gamma + beta, 0.0)


def _film_tc(xp, cond2, vt, g2, b2, wt, bf2):
  grid = NPAD // BN
  return pl.pallas_call(
      _film_body,
      grid=(grid,),
      in_specs=[
          pl.BlockSpec((1, C), lambda i: (0, 0)),
          pl.BlockSpec((C, 2 * O), lambda i: (0, 0)),
          pl.BlockSpec((1, 2 * O), lambda i: (0, 0)),
          pl.BlockSpec((1, 2 * O), lambda i: (0, 0)),
          pl.BlockSpec((BN, D), lambda i: (i, 0)),
          pl.BlockSpec((D, O), lambda i: (0, 0)),
          pl.BlockSpec((1, O), lambda i: (0, 0)),
      ],
      out_specs=pl.BlockSpec((BN, O), lambda i: (i, 0)),
      out_shape=jax.ShapeDtypeStruct((NPAD, O), jnp.float32),
      scratch_shapes=[pltpu.VMEM((1, 2 * O), jnp.float32)],
  )(cond2, vt, g2, b2, xp, wt, bf2)


_SC_MESH = plsc.VectorSubcoreMesh(core_axis_name="c", subcore_axis_name="s")


@functools.partial(
    pl.kernel,
    out_type=jax.ShapeDtypeStruct((NPAD, O), jnp.float32),
    mesh=_SC_MESH,
    scratch_types=[
        pltpu.VMEM((NCH, CHUNK), jnp.int32),
        pltpu.VMEM((NCH, CHUNK), jnp.float32),
        pltpu.VMEM((NCH, CHUNK), jnp.float32),
        pltpu.VMEM((CHUNK, O), jnp.float32),
        pltpu.VMEM((CHUNK, O), jnp.float32),
        pltpu.VMEM((NPW, O), jnp.float32),
        pltpu.SemaphoreType.DMA,
        pltpu.SemaphoreType.DMA,
    ],
)
def _agg_sc(feats_hbm, idx_hbm, wv_hbm, wp_hbm, out_hbm,
            idx_v, wv_v, wp_v, rows0, rows1, out_v, sem0, sem1):
  nc = _SC_MESH.num_cores
  wid = lax.axis_index("s") * nc + lax.axis_index("c")
  base = wid * NPW

  pltpu.sync_copy(idx_hbm.at[wid], idx_v)
  pltpu.sync_copy(wv_hbm.at[wid], wv_v)
  pltpu.sync_copy(wp_hbm.at[wid], wp_v)
  pltpu.sync_copy(feats_hbm.at[pl.ds(base, NPW)], out_v)

  def _wbody(r, carry):
    for i in range(CHUNK // 16):
      s = pl.ds(i * 16, 16)
      wv_v[r, s] = wv_v[r, s] * wp_v[r, s]
    return carry
  lax.fori_loop(0, NCH, _wbody, 0)

  def _start(c, buf, sem):
    pltpu.async_copy(feats_hbm.at[idx_v.at[c]], buf, sem)

  def _wait(buf, sem):
    pltpu.make_async_copy(feats_hbm.at[idx_v.at[0]], buf, sem).wait()

  def _compute(c, buf):
    for n in range(CN):
      row = c * CN + n
      acc = tuple(out_v[row, pl.ds(d * 16, 16)] for d in range(NV))

      def _kbody(k, a):
        e = n * K + k
        w = lax.broadcast(wv_v[c, e], (16,))
        return tuple(a[d] + w * buf[e, pl.ds(d * 16, 16)]
                     for d in range(NV))

      acc = lax.fori_loop(0, K, _kbody, acc)
      for d in range(NV):
        out_v[row, pl.ds(d * 16, 16)] = jnp.maximum(acc[d], 0.0)

  _start(0, rows0, sem0)

  def _body(i, carry):
    c0 = 2 * i
    c1 = c0 + 1
    _start(c1, rows1, sem1)
    _wait(rows0, sem0)
    _compute(c0, rows0)

    @pl.when(c0 + 2 < NCH)
    def _():
      _start(c0 + 2, rows0, sem0)

    _wait(rows1, sem1)
    _compute(c1, rows1)
    return carry

  lax.fori_loop(0, NCH // 2, _body, 0)

  pltpu.sync_copy(out_v, out_hbm.at[pl.ds(base, NPW)])


def kernel(x, cond, idx_j, w_val, w_param, v_cond, g_cond, b_cond,
           W_film, b_film):
  xp = jnp.pad(x.reshape(N, D), ((0, NPAD - N), (0, 0)))
  feats = _film_tc(
      xp,
      cond.reshape(1, C),
      v_cond.T,
      g_cond.reshape(1, 2 * O),
      b_cond.reshape(1, 2 * O),
      W_film.T,
      b_film.reshape(1, O),
  )

  pad_e = (NPAD - N) * K
  idxp = jnp.pad(idx_j.astype(jnp.int32), (0, pad_e)).reshape(NW, NCH, CHUNK)
  wv = jnp.pad(w_val.reshape(N * K), (0, pad_e)).reshape(NW, NCH, CHUNK)
  wp = jnp.pad(w_param.reshape(N * K), (0, pad_e)).reshape(NW, NCH, CHUNK)

  out = _agg_sc(feats, idxp, wv, wp)
  return out[:N].reshape(1, N, O)


# speedup vs baseline: 1.7583x; 1.7583x over previous
import functools

import jax
import jax.numpy as jnp
from jax import lax
from jax.experimental import pallas as pl
from jax.experimental.pallas import tpu as pltpu
from jax.experimental.pallas import tpu_sc as plsc

N = 10000
K = 32
D = 128
O = 128
C = 128

NW = 32
NPW = 320
NPAD = NW * NPW
CN = 4
CHUNK = CN * K
NCH = NPW // CN
NV = O // 16

BN = 256


def _film_body(cond_ref, vt_ref, g_ref, b2_ref, x_ref, wt_ref, bf_ref,
               o_ref, gb_ref):
  @pl.when(pl.program_id(0) == 0)
  def _():
    vt = vt_ref[...]
    ssq = jnp.sum(vt * vt, axis=0, keepdims=True)
    scale = g_ref[...] * lax.rsqrt(ssq)
    gb_ref[...] = (
        jnp.dot(cond_ref[...], vt, preferred_element_type=jnp.float32,
                precision=lax.Precision.HIGHEST) * scale + b2_ref[...])

  h = jnp.dot(x_ref[...], wt_ref[...], preferred_element_type=jnp.float32,
              precision=lax.Precision.HIGHEST) + bf_ref[...]
  mu = jnp.mean(h, axis=1, keepdims=True)
  hc = h - mu
  var = jnp.mean(hc * hc, axis=1, keepdims=True)
  hn = hc * lax.rsqrt(var + 1e-5)
  gamma = gb_ref[:, :O] + 1.0
  beta = gb_ref[:, O:]
  o_ref[...] = jnp.maximum(hn * gamma + beta, 0.0)


def _film_tc(xp, cond2, vt, g2, b2, wt, bf2):
  grid = NPAD // BN
  return pl.pallas_call(
      _film_body,
      grid=(grid,),
      in_specs=[
          pl.BlockSpec((1, C), lambda i: (0, 0)),
          pl.BlockSpec((C, 2 * O), lambda i: (0, 0)),
          pl.BlockSpec((1, 2 * O), lambda i: (0, 0)),
          pl.BlockSpec((1, 2 * O), lambda i: (0, 0)),
          pl.BlockSpec((BN, D), lambda i: (i, 0)),
          pl.BlockSpec((D, O), lambda i: (0, 0)),
          pl.BlockSpec((1, O), lambda i: (0, 0)),
      ],
      out_specs=pl.BlockSpec((BN, O), lambda i: (i, 0)),
      out_shape=jax.ShapeDtypeStruct((NPAD, O), jnp.float32),
      scratch_shapes=[pltpu.VMEM((1, 2 * O), jnp.float32)],
  )(cond2, vt, g2, b2, xp, wt, bf2)


_SC_MESH = plsc.VectorSubcoreMesh(core_axis_name="c", subcore_axis_name="s")


@functools.partial(
    pl.kernel,
    out_type=jax.ShapeDtypeStruct((NPAD, O), jnp.float32),
    mesh=_SC_MESH,
    scratch_types=[
        pltpu.VMEM((NCH, CHUNK), jnp.int32),
        pltpu.VMEM((NCH * CHUNK + 16,), jnp.float32),
        pltpu.VMEM((NCH * CHUNK,), jnp.float32),
        pltpu.VMEM((CHUNK, O), jnp.float32),
        pltpu.VMEM((CHUNK, O), jnp.float32),
        pltpu.VMEM((NPW, O), jnp.float32),
        pltpu.SemaphoreType.DMA,
        pltpu.SemaphoreType.DMA,
    ],
)
def _agg_sc(feats_hbm, idx_hbm, wv_hbm, wp_hbm, out_hbm,
            idx_v, wv_v, wp_v, rows0, rows1, out_v, sem0, sem1):
  nc = _SC_MESH.num_cores
  wid = lax.axis_index("s") * nc + lax.axis_index("c")
  base = wid * NPW

  pltpu.sync_copy(idx_hbm.at[wid], idx_v)
  pltpu.sync_copy(wv_hbm.at[wid], wv_v.at[pl.ds(0, NCH * CHUNK)])
  pltpu.sync_copy(wp_hbm.at[wid], wp_v)
  pltpu.sync_copy(feats_hbm.at[pl.ds(base, NPW)], out_v)

  def _wbody(r, carry):
    s = pl.ds(r * 16, 16)
    wv_v[s] = wv_v[s] * wp_v[s]
    return carry
  lax.fori_loop(0, NCH * CHUNK // 16, _wbody, 0)

  def _start(c, buf, sem):
    pltpu.async_copy(feats_hbm.at[idx_v.at[c]], buf, sem)

  def _wait(buf, sem):
    pltpu.make_async_copy(feats_hbm.at[idx_v.at[0]], buf, sem).wait()

  def _compute(c, buf):
    for n in range(CN):
      row = c * CN + n
      acc = tuple(out_v[row, pl.ds(d * 16, 16)] for d in range(NV))

      def _kbody(k, a):
        e = n * K + k
        w = lax.broadcast(wv_v[pl.ds(c * CHUNK + e, 16)][0], (16,))
        return tuple(a[d] + w * buf[e, pl.ds(d * 16, 16)]
                     for d in range(NV))

      acc = lax.fori_loop(0, K, _kbody, acc)
      for d in range(NV):
        out_v[row, pl.ds(d * 16, 16)] = jnp.maximum(acc[d], 0.0)

  _start(0, rows0, sem0)

  def _body(i, carry):
    c0 = 2 * i
    c1 = c0 + 1
    _start(c1, rows1, sem1)
    _wait(rows0, sem0)
    _compute(c0, rows0)

    @pl.when(c0 + 2 < NCH)
    def _():
      _start(c0 + 2, rows0, sem0)

    _wait(rows1, sem1)
    _compute(c1, rows1)
    return carry

  lax.fori_loop(0, NCH // 2, _body, 0)

  pltpu.sync_copy(out_v, out_hbm.at[pl.ds(base, NPW)])


def kernel(x, cond, idx_j, w_val, w_param, v_cond, g_cond, b_cond,
           W_film, b_film):
  xp = jnp.pad(x.reshape(N, D), ((0, NPAD - N), (0, 0)))
  feats = _film_tc(
      xp,
      cond.reshape(1, C),
      v_cond.T,
      g_cond.reshape(1, 2 * O),
      b_cond.reshape(1, 2 * O),
      W_film.T,
      b_film.reshape(1, O),
  )

  pad_e = (NPAD - N) * K
  idxp = jnp.pad(idx_j.astype(jnp.int32), (0, pad_e)).reshape(NW, NCH, CHUNK)
  wv = jnp.pad(w_val.reshape(N * K), (0, pad_e)).reshape(NW, NCH * CHUNK)
  wp = jnp.pad(w_param.reshape(N * K), (0, pad_e)).reshape(NW, NCH * CHUNK)

  out = _agg_sc(feats, idxp, wv, wp)
  return out[:N].reshape(1, N, O)
